# baseline (speedup 1.0000x reference)
import functools

import jax
import jax.numpy as jnp
from jax import lax
from jax.experimental import pallas as pl
from jax.experimental.pallas import tpu as pltpu
from jax.experimental.pallas import tpu_sc as plsc

MAX_SEQ_LEN = 8192
D_MODEL = 1024
BATCH = 4
SEQ = 8192
B_TOTAL = BATCH * SEQ

NUM_CORES = 2
NUM_SUBCORES = 16
NW = NUM_CORES * NUM_SUBCORES
B_PER_W = B_TOTAL // NW

NBUF = 3
REG = 32
GCH = 8
GPER = REG // GCH
NREG = B_PER_W // REG
NGRP = 9

_mesh = plsc.VectorSubcoreMesh(core_axis_name="c", subcore_axis_name="s")


@functools.partial(
    pl.kernel,
    mesh=_mesh,
    out_type=jax.ShapeDtypeStruct((B_TOTAL, D_MODEL), jnp.float32),
    scratch_types=[
        pltpu.VMEM((B_PER_W,), jnp.int32),
    ]
    + [pltpu.VMEM((REG, D_MODEL), jnp.float32) for _ in range(NBUF)]
    + [pltpu.SemaphoreType.DMA for _ in range(2 * NBUF)],
)
def _gather_rows(table_hbm, idx_hbm, out_hbm, idx_v, *bufs_and_sems):
    bufs = bufs_and_sems[:NBUF]
    gsems = bufs_and_sems[NBUF : 2 * NBUF]
    wsems = bufs_and_sems[2 * NBUF : 3 * NBUF]

    wid = lax.axis_index("s") * NUM_CORES + lax.axis_index("c")
    base = wid * B_PER_W
    pltpu.sync_copy(idx_hbm.at[pl.ds(base, B_PER_W)], idx_v)

    def start_gather(reg, r):
        for g in range(GPER):
            idx_slice = idx_v.at[pl.ds(reg * REG + g * GCH, GCH)]
            pltpu.async_copy(
                table_hbm.at[idx_slice], bufs[r].at[pl.ds(g * GCH, GCH)], gsems[r]
            )

    def wait_gather(r):
        pltpu.make_async_copy(table_hbm.at[pl.ds(0, REG)], bufs[r], gsems[r]).wait()

    def start_write(reg, r):
        pltpu.async_copy(bufs[r], out_hbm.at[pl.ds(base + reg * REG, REG)], wsems[r])

    def wait_write(r):
        pltpu.make_async_copy(bufs[r], out_hbm.at[pl.ds(base, REG)], wsems[r]).wait()

    for r in range(NBUF):
        start_gather(r, r)

    def body(grp, _):
        r0 = grp * NBUF
        for r in range(NBUF):
            wait_gather(r)
            start_write(r0 + r, r)
        for r in range(NBUF):
            wait_write(r)
            start_gather(r0 + NBUF + r, r)
        return ()

    lax.fori_loop(0, NGRP, body, ())

    r0 = NGRP * NBUF
    for r in range(NBUF):
        wait_gather(r)
        start_write(r0 + r, r)
    for r in range(2):
        wait_write(r)
        start_gather(r0 + NBUF + r, r)
    wait_write(2)
    for r in range(2):
        wait_gather(r)
        start_write(r0 + NBUF + r, r)
    for r in range(2):
        wait_write(r)


def kernel(pos_table, src_seq):
    flat_idx = src_seq.reshape(-1).astype(jnp.int32)
    out = _gather_rows(pos_table, flat_idx)
    return out.reshape(BATCH, SEQ, D_MODEL)
